# baseline (speedup 1.0000x reference)
import math

import jax
import jax.numpy as jnp
from jax.experimental import pallas as pl
from jax.experimental.pallas import tpu as pltpu

EMB_DIM = 1024
HALF_DIM = EMB_DIM // 2
PAD_IDX = 0
BLK = 256
NSLOT = 5


def _sinusoid_kernel(out_ref, buf, base_sin, base_cos, sem):
    pid = pl.program_id(0)
    nblk = pl.num_programs(0)
    bsz = out_ref.shape[0]
    scale = math.log(10000.0) / (HALF_DIM - 1)

    @pl.when(pid == 0)
    def _init():
        row = jax.lax.broadcasted_iota(jnp.int32, (8, HALF_DIM), 0)
        col = jax.lax.broadcasted_iota(jnp.int32, (8, HALF_DIM), 1)
        freq = jnp.exp(col.astype(jnp.float32) * jnp.float32(-scale))
        phase = (row.astype(jnp.float32) + jnp.float32(PAD_IDX + 1)) * freq
        base_sin[0:8] = jnp.sin(phase)
        base_cos[0:8] = jnp.cos(phase)
        mrow = jnp.exp2(row.astype(jnp.float32) + jnp.float32(3.0))
        shift_phase = mrow * freq
        shift_sin = jnp.sin(shift_phase)
        shift_cos = jnp.cos(shift_phase)
        m = 8
        k = 0
        while m < BLK:
            s_b = shift_sin[k : k + 1]
            c_b = shift_cos[k : k + 1]
            s = base_sin[0:m]
            c = base_cos[0:m]
            base_sin[m : 2 * m] = s * c_b + c * s_b
            base_cos[m : 2 * m] = c * c_b - s * s_b
            m *= 2
            k += 1

    slot = jax.lax.rem(pid, NSLOT)

    def _copy(s, b, dst_start):
        return pltpu.make_async_copy(
            buf.at[s],
            out_ref.at[b, pl.ds(dst_start, BLK), :],
            sem.at[s],
        )

    @pl.when(pid >= NSLOT)
    def _wait_prev():
        for b in range(bsz):
            _copy(slot, b, (pid - NSLOT) * BLK).wait()

    colr = jax.lax.broadcasted_iota(jnp.int32, (8, HALF_DIM), 1).astype(jnp.float32)
    freqr = jnp.exp(colr * jnp.float32(-scale))
    shift = (pid * BLK).astype(jnp.float32) * freqr
    sin_b = jnp.sin(shift)[:1]
    cos_b = jnp.cos(shift)[:1]

    s_a = base_sin[...]
    c_a = base_cos[...]
    out_sin = s_a * cos_b + c_a * sin_b
    out_cos = c_a * cos_b - s_a * sin_b
    buf[slot] = jnp.concatenate([out_sin, out_cos], axis=1)

    for b in range(bsz):
        _copy(slot, b, pid * BLK).start()

    @pl.when(pid == nblk - 1)
    def _drain():
        for s in range(NSLOT):
            last_pid = nblk - NSLOT + s
            sl = jax.lax.rem(jnp.int32(last_pid), NSLOT)
            for b in range(bsz):
                _copy(sl, b, last_pid * BLK).wait()


def kernel(input):
    bsz, seqlen = input.shape
    grid = (seqlen // BLK,)
    out = pl.pallas_call(
        _sinusoid_kernel,
        grid=grid,
        out_specs=pl.BlockSpec(memory_space=pltpu.HBM),
        out_shape=jax.ShapeDtypeStruct((bsz, seqlen, EMB_DIM), input.dtype),
        scratch_shapes=[
            pltpu.VMEM((NSLOT, BLK, EMB_DIM), jnp.float32),
            pltpu.VMEM((BLK, HALF_DIM), jnp.float32),
            pltpu.VMEM((BLK, HALF_DIM), jnp.float32),
            pltpu.SemaphoreType.DMA((NSLOT,)),
        ],
    )()
    return out


# speedup vs baseline: 1.0626x; 1.0626x over previous
import math

import jax
import jax.numpy as jnp
from jax.experimental import pallas as pl
from jax.experimental.pallas import tpu as pltpu

EMB_DIM = 1024
HALF_DIM = EMB_DIM // 2
PAD_IDX = 0
BLK = 256


def _sinusoid_kernel(out_ref, base_sin, base_cos):
    pid = pl.program_id(0)
    scale = math.log(10000.0) / (HALF_DIM - 1)

    @pl.when(pid == 0)
    def _init():
        row = jax.lax.broadcasted_iota(jnp.int32, (8, HALF_DIM), 0)
        col = jax.lax.broadcasted_iota(jnp.int32, (8, HALF_DIM), 1)
        freq = jnp.exp(col.astype(jnp.float32) * jnp.float32(-scale))
        phase = (row.astype(jnp.float32) + jnp.float32(PAD_IDX + 1)) * freq
        base_sin[0:8] = jnp.sin(phase)
        base_cos[0:8] = jnp.cos(phase)
        mrow = jnp.exp2(row.astype(jnp.float32) + jnp.float32(3.0))
        shift_phase = mrow * freq
        shift_sin = jnp.sin(shift_phase)
        shift_cos = jnp.cos(shift_phase)
        m = 8
        k = 0
        while m < BLK:
            s_b = shift_sin[k : k + 1]
            c_b = shift_cos[k : k + 1]
            s = base_sin[0:m]
            c = base_cos[0:m]
            base_sin[m : 2 * m] = s * c_b + c * s_b
            base_cos[m : 2 * m] = c * c_b - s * s_b
            m *= 2
            k += 1

    colr = jax.lax.broadcasted_iota(jnp.int32, (8, HALF_DIM), 1).astype(jnp.float32)
    freqr = jnp.exp(colr * jnp.float32(-scale))
    shift = (pid * BLK).astype(jnp.float32) * freqr
    sin_b = jnp.sin(shift)[:1]
    cos_b = jnp.cos(shift)[:1]

    s_a = base_sin[...]
    c_a = base_cos[...]
    out_sin = s_a * cos_b + c_a * sin_b
    out_cos = c_a * cos_b - s_a * sin_b
    block = jnp.concatenate([out_sin, out_cos], axis=1)
    out_ref[...] = jnp.broadcast_to(block[None], out_ref.shape)


def kernel(input):
    bsz, seqlen = input.shape
    grid = (seqlen // BLK,)
    out = pl.pallas_call(
        _sinusoid_kernel,
        grid=grid,
        out_specs=pl.BlockSpec((bsz, BLK, EMB_DIM), lambda i: (0, i, 0)),
        out_shape=jax.ShapeDtypeStruct((bsz, seqlen, EMB_DIM), input.dtype),
        scratch_shapes=[
            pltpu.VMEM((BLK, HALF_DIM), jnp.float32),
            pltpu.VMEM((BLK, HALF_DIM), jnp.float32),
        ],
    )()
    return out
